# Initial kernel scaffold; baseline (speedup 1.0000x reference)
import jax
import jax.numpy as jnp
from jax.experimental import pallas as pl


def kernel(x, y, embedding):
    raise NotImplementedError("write your pallas kernel here")


# speedup vs baseline: 1.0943x; 1.0943x over previous
import functools

import jax
import jax.numpy as jnp
from jax import lax
from jax.experimental import pallas as pl
from jax.experimental.pallas import tpu as pltpu
from jax.experimental.pallas import tpu_sc as plsc

DIM = 32
IDX_ROW = 128
ROWS_PER_CHUNK = 1024
IDX_ROWS_PER_CHUNK = ROWS_PER_CHUNK // IDX_ROW


def _gather_kernel(b_total, num_workers):
    b_per_w = b_total // num_workers
    n_chunks = b_per_w // ROWS_PER_CHUNK
    mesh = plsc.VectorSubcoreMesh(core_axis_name="c", subcore_axis_name="s")
    nc = plsc.get_sparse_core_info().num_cores

    @functools.partial(
        pl.kernel,
        mesh=mesh,
        out_type=jax.ShapeDtypeStruct((b_total, DIM), jnp.float32),
        compiler_params=pltpu.CompilerParams(use_tc_tiling_on_sc=False),
        scratch_types=[
            pltpu.VMEM((IDX_ROWS_PER_CHUNK, IDX_ROW), jnp.int32),
            pltpu.VMEM((ROWS_PER_CHUNK, DIM), jnp.float32),
            pltpu.SemaphoreType.DMA,
        ],
    )
    def k(table_hbm, idx_hbm, out_hbm, idx_v, rows_v, sem):
        wid = lax.axis_index("s") * nc + lax.axis_index("c")
        base = wid * b_per_w

        def body(i, _):
            off = base + i * ROWS_PER_CHUNK
            irow = pl.multiple_of(off // IDX_ROW, 8)
            pltpu.sync_copy(idx_hbm.at[pl.ds(irow, IDX_ROWS_PER_CHUNK)], idx_v)
            copies = []
            for j in range(IDX_ROWS_PER_CHUNK):
                copies.append(
                    pltpu.async_copy(
                        table_hbm.at[idx_v.at[j]],
                        rows_v.at[pl.ds(j * IDX_ROW, IDX_ROW)],
                        sem,
                    )
                )
            for c in copies:
                c.wait()
            pltpu.sync_copy(rows_v, out_hbm.at[pl.ds(off, ROWS_PER_CHUNK)])
            return 0

        lax.fori_loop(0, n_chunks, body, 0)

    return k


def kernel(x, y, embedding):
    b, s = x.shape
    idx = x.reshape(-1).astype(jnp.int32).reshape(-1, IDX_ROW)
    out = _gather_kernel(b * s, 32)(embedding, idx)
    return out.reshape(b, s, DIM)


# speedup vs baseline: 1.1108x; 1.0152x over previous
import functools

import jax
import jax.numpy as jnp
from jax import lax
from jax.experimental import pallas as pl
from jax.experimental.pallas import tpu as pltpu
from jax.experimental.pallas import tpu_sc as plsc

DIM = 32
IDX_PER_DMA = 128
CHUNK = 1280
K = CHUNK // IDX_PER_DMA


def _gather_kernel(b_total, num_workers):
    b_per_w = b_total // num_workers
    n_chunks = b_per_w // CHUNK
    assert n_chunks % 2 == 0 and n_chunks >= 4
    n_pairs = (n_chunks - 2) // 2
    mesh = plsc.VectorSubcoreMesh(core_axis_name="c", subcore_axis_name="s")
    nc = plsc.get_sparse_core_info().num_cores

    @functools.partial(
        pl.kernel,
        mesh=mesh,
        out_type=jax.ShapeDtypeStruct((b_total, DIM), jnp.float32),
        compiler_params=pltpu.CompilerParams(use_tc_tiling_on_sc=False),
        scratch_types=[
            pltpu.VMEM((CHUNK,), jnp.int32),
            pltpu.VMEM((CHUNK,), jnp.int32),
            pltpu.VMEM((CHUNK, DIM), jnp.float32),
            pltpu.VMEM((CHUNK, DIM), jnp.float32),
            pltpu.SemaphoreType.DMA,
            pltpu.SemaphoreType.DMA,
            pltpu.SemaphoreType.DMA,
            pltpu.SemaphoreType.DMA,
        ],
    )
    def k(table_hbm, idx_hbm, out_hbm, idx0, idx1, rows0, rows1,
          gsem0, gsem1, osem0, osem1):
        wid = lax.axis_index("s") * nc + lax.axis_index("c")
        base = wid * b_per_w

        def off_of(i):
            return pl.multiple_of(base + i * CHUNK, CHUNK)

        def stage_idx(i, idxbuf):
            pltpu.sync_copy(idx_hbm.at[pl.ds(off_of(i), CHUNK)], idxbuf)

        def fire_gathers(idxbuf, rowsbuf, sem):
            for j in range(K):
                pltpu.async_copy(
                    table_hbm.at[idxbuf.at[pl.ds(j * IDX_PER_DMA, IDX_PER_DMA)]],
                    rowsbuf.at[pl.ds(j * IDX_PER_DMA, IDX_PER_DMA)],
                    sem,
                )

        def wait_gathers(rowsbuf, sem):
            pltpu.make_async_copy(out_hbm.at[pl.ds(0, CHUNK)], rowsbuf, sem).wait()

        def fire_out(i, rowsbuf, sem):
            pltpu.async_copy(rowsbuf, out_hbm.at[pl.ds(off_of(i), CHUNK)], sem)

        def wait_out(rowsbuf, sem):
            pltpu.make_async_copy(rowsbuf, out_hbm.at[pl.ds(0, CHUNK)], sem).wait()

        stage_idx(0, idx0)
        fire_gathers(idx0, rows0, gsem0)
        stage_idx(1, idx1)
        fire_gathers(idx1, rows1, gsem1)

        def body(t, _):
            a = 2 * t
            wait_gathers(rows0, gsem0)
            fire_out(a, rows0, osem0)
            wait_gathers(rows1, gsem1)
            fire_out(a + 1, rows1, osem1)
            stage_idx(a + 2, idx0)
            wait_out(rows0, osem0)
            fire_gathers(idx0, rows0, gsem0)
            stage_idx(a + 3, idx1)
            wait_out(rows1, osem1)
            fire_gathers(idx1, rows1, gsem1)
            return 0

        lax.fori_loop(0, n_pairs, body, 0, unroll=False)

        last = n_chunks - 2
        wait_gathers(rows0, gsem0)
        fire_out(last, rows0, osem0)
        wait_gathers(rows1, gsem1)
        fire_out(last + 1, rows1, osem1)
        wait_out(rows0, osem0)
        wait_out(rows1, osem1)

    return k


def kernel(x, y, embedding):
    b, s = x.shape
    idx = x.reshape(-1).astype(jnp.int32)
    out = _gather_kernel(b * s, 32)(embedding, idx)
    return out.reshape(b, s, DIM)


# speedup vs baseline: 1.4640x; 1.3179x over previous
import functools

import jax
import jax.numpy as jnp
from jax import lax
from jax.experimental import pallas as pl
from jax.experimental.pallas import tpu as pltpu
from jax.experimental.pallas import tpu_sc as plsc

DIM = 32
IDX_PER_DMA = 128
CHUNK = 512
K = CHUNK // IDX_PER_DMA
NW = 32


def _gather_kernel(seq, batch):
    n_chunks = seq
    assert n_chunks % 2 == 0
    n_pairs = (n_chunks - 2) // 2
    mesh = plsc.VectorSubcoreMesh(core_axis_name="c", subcore_axis_name="s")
    nc = plsc.get_sparse_core_info().num_cores

    @functools.partial(
        pl.kernel,
        mesh=mesh,
        out_type=jax.ShapeDtypeStruct((seq, DIM, batch), jnp.float32),
        compiler_params=pltpu.CompilerParams(
            use_tc_tiling_on_sc=False, needs_layout_passes=False
        ),
        scratch_types=[
            pltpu.VMEM((CHUNK,), jnp.int32),
            pltpu.VMEM((CHUNK,), jnp.int32),
            pltpu.VMEM((CHUNK, DIM), jnp.float32),
            pltpu.VMEM((CHUNK, DIM), jnp.float32),
            pltpu.VMEM((DIM, CHUNK), jnp.float32),
            pltpu.VMEM((DIM, CHUNK), jnp.float32),
            pltpu.SemaphoreType.DMA,
            pltpu.SemaphoreType.DMA,
            pltpu.SemaphoreType.DMA,
            pltpu.SemaphoreType.DMA,
        ],
    )
    def k(table_hbm, idx_hbm, out_hbm, idx0, idx1, rows0, rows1, t0, t1,
          gsem0, gsem1, osem0, osem1):
        wid = lax.axis_index("s") * nc + lax.axis_index("c")
        b0 = pl.multiple_of(wid * CHUNK, CHUNK)
        lane_row = lax.iota(jnp.int32, 16)

        def stage_idx(s, idxbuf):
            off = pl.multiple_of(s * batch + b0, CHUNK)
            pltpu.sync_copy(idx_hbm.at[pl.ds(off, CHUNK)], idxbuf)

        def fire_gathers(idxbuf, rowsbuf, sem):
            for j in range(K):
                pltpu.async_copy(
                    table_hbm.at[idxbuf.at[pl.ds(j * IDX_PER_DMA, IDX_PER_DMA)]],
                    rowsbuf.at[pl.ds(j * IDX_PER_DMA, IDX_PER_DMA)],
                    sem,
                )

        def wait_gathers(rowsbuf, sem):
            pltpu.make_async_copy(
                out_hbm.at[0, :, pl.ds(0, CHUNK)], rowsbuf, sem
            ).wait()

        def transpose(rowsbuf, tbuf):
            def g_body(g, _):
                rbase = g * 16
                rows16 = lane_row + rbase
                for d in range(DIM):
                    vals = plsc.load_gather(
                        rowsbuf, [rows16, jnp.full((16,), d, jnp.int32)]
                    )
                    tbuf[d, pl.ds(rbase, 16)] = vals
                return 0

            lax.fori_loop(0, CHUNK // 16, g_body, 0, unroll=False)

        def fire_out(s, tbuf, sem):
            pltpu.async_copy(tbuf, out_hbm.at[s, :, pl.ds(b0, CHUNK)], sem)

        def wait_out(tbuf, sem):
            pltpu.make_async_copy(
                tbuf, out_hbm.at[0, :, pl.ds(0, CHUNK)], sem
            ).wait()

        stage_idx(0, idx0)
        fire_gathers(idx0, rows0, gsem0)
        stage_idx(1, idx1)
        fire_gathers(idx1, rows1, gsem1)

        def body(t, _):
            a = 2 * t
            wait_gathers(rows0, gsem0)
            wait_out(t0, osem0)
            transpose(rows0, t0)
            fire_out(a, t0, osem0)
            stage_idx(a + 2, idx0)
            fire_gathers(idx0, rows0, gsem0)
            wait_gathers(rows1, gsem1)
            wait_out(t1, osem1)
            transpose(rows1, t1)
            fire_out(a + 1, t1, osem1)
            stage_idx(a + 3, idx1)
            fire_gathers(idx1, rows1, gsem1)
            return 0

        wait_gathers(rows0, gsem0)
        transpose(rows0, t0)
        fire_out(0, t0, osem0)
        stage_idx(2, idx0)
        fire_gathers(idx0, rows0, gsem0)
        wait_gathers(rows1, gsem1)
        transpose(rows1, t1)
        fire_out(1, t1, osem1)
        stage_idx(3, idx1)
        fire_gathers(idx1, rows1, gsem1)

        def body_shifted(t, _):
            return body(t + 1, None)

        lax.fori_loop(0, n_pairs - 1, body_shifted, 0, unroll=False)

        last = n_chunks - 2
        wait_gathers(rows0, gsem0)
        wait_out(t0, osem0)
        transpose(rows0, t0)
        fire_out(last, t0, osem0)
        wait_gathers(rows1, gsem1)
        wait_out(t1, osem1)
        transpose(rows1, t1)
        fire_out(last + 1, t1, osem1)
        wait_out(t0, osem0)
        wait_out(t1, osem1)

    return k


def kernel(x, y, embedding):
    b, s = x.shape
    idx = x.T.reshape(-1).astype(jnp.int32)
    out_p = _gather_kernel(s, b)(embedding, idx)
    return jnp.transpose(out_p, (2, 0, 1))


# speedup vs baseline: 2.2511x; 1.5377x over previous
import functools

import jax
import jax.numpy as jnp
from jax import lax
from jax.experimental import pallas as pl
from jax.experimental.pallas import tpu as pltpu
from jax.experimental.pallas import tpu_sc as plsc

DIM = 32
IDX_PER_DMA = 128
CHUNK = 512
K = CHUNK // IDX_PER_DMA
LANES = 16


def _gather_kernel(seq, batch):
    n_chunks = seq
    assert n_chunks % 2 == 0
    n_pairs = n_chunks // 2
    mesh = plsc.VectorSubcoreMesh(core_axis_name="c", subcore_axis_name="s")
    nc = plsc.get_sparse_core_info().num_cores

    @functools.partial(
        pl.kernel,
        mesh=mesh,
        out_type=jax.ShapeDtypeStruct((seq, DIM, batch), jnp.float32),
        compiler_params=pltpu.CompilerParams(
            use_tc_tiling_on_sc=False, needs_layout_passes=False
        ),
        scratch_types=[
            pltpu.VMEM((seq, CHUNK), jnp.int32),
            pltpu.VMEM((CHUNK, DIM), jnp.float32),
            pltpu.VMEM((CHUNK, DIM), jnp.float32),
            pltpu.VMEM((DIM, CHUNK), jnp.float32),
            pltpu.VMEM((DIM, CHUNK), jnp.float32),
            pltpu.SemaphoreType.DMA,
            pltpu.SemaphoreType.DMA,
            pltpu.SemaphoreType.DMA,
            pltpu.SemaphoreType.DMA,
            pltpu.SemaphoreType.DMA,
        ],
    )
    def k(table_hbm, xt_hbm, out_hbm, xbuf, rows0, rows1, t0, t1,
          xsem, gsem0, gsem1, osem0, osem1):
        wid = lax.axis_index("s") * nc + lax.axis_index("c")
        b0 = pl.multiple_of(wid * CHUNK, CHUNK)
        lane = lax.iota(jnp.int32, LANES)
        diag_cols = [(lane + d0) & (DIM - 1) for d0 in range(DIM)]

        pltpu.async_copy(xt_hbm.at[:, pl.ds(b0, CHUNK)], xbuf, xsem).wait()

        def fire_gathers(s, rowsbuf, sem):
            for j in range(K):
                pltpu.async_copy(
                    table_hbm.at[xbuf.at[s, pl.ds(j * IDX_PER_DMA, IDX_PER_DMA)]],
                    rowsbuf.at[pl.ds(j * IDX_PER_DMA, IDX_PER_DMA)],
                    sem,
                )

        def wait_gathers(rowsbuf, sem):
            pltpu.make_async_copy(
                out_hbm.at[0, :, pl.ds(0, CHUNK)], rowsbuf, sem
            ).wait()

        def transpose(rowsbuf, tbuf):
            def g_body(g, _):
                rows16 = lane + g * LANES
                for d0 in range(DIM):
                    cols = diag_cols[d0]
                    vals = plsc.load_gather(rowsbuf, [rows16, cols])
                    plsc.store_scatter(tbuf, [cols, rows16], vals)
                return 0

            lax.fori_loop(0, CHUNK // LANES, g_body, 0, unroll=False)

        def fire_out(s, tbuf, sem):
            pltpu.async_copy(tbuf, out_hbm.at[s, :, pl.ds(b0, CHUNK)], sem)

        def wait_out(tbuf, sem):
            pltpu.make_async_copy(
                tbuf, out_hbm.at[0, :, pl.ds(0, CHUNK)], sem
            ).wait()

        fire_gathers(0, rows0, gsem0)
        fire_gathers(1, rows1, gsem1)

        wait_gathers(rows0, gsem0)
        transpose(rows0, t0)
        fire_out(0, t0, osem0)
        fire_gathers(2, rows0, gsem0)
        wait_gathers(rows1, gsem1)
        transpose(rows1, t1)
        fire_out(1, t1, osem1)
        fire_gathers(3, rows1, gsem1)

        def body(t, _):
            a = 2 * t
            wait_gathers(rows0, gsem0)
            wait_out(t0, osem0)
            transpose(rows0, t0)
            fire_out(a, t0, osem0)
            fire_gathers(a + 2, rows0, gsem0)
            wait_gathers(rows1, gsem1)
            wait_out(t1, osem1)
            transpose(rows1, t1)
            fire_out(a + 1, t1, osem1)
            fire_gathers(a + 3, rows1, gsem1)
            return 0

        lax.fori_loop(1, n_pairs - 1, body, 0, unroll=False)

        last = n_chunks - 2
        wait_gathers(rows0, gsem0)
        wait_out(t0, osem0)
        transpose(rows0, t0)
        fire_out(last, t0, osem0)
        wait_gathers(rows1, gsem1)
        wait_out(t1, osem1)
        transpose(rows1, t1)
        fire_out(last + 1, t1, osem1)
        wait_out(t0, osem0)
        wait_out(t1, osem1)

    return k


def kernel(x, y, embedding):
    b, s = x.shape
    out_p = _gather_kernel(s, b)(embedding, x.T.astype(jnp.int32))
    return jnp.transpose(out_p, (2, 0, 1))


# speedup vs baseline: 2.2720x; 1.0093x over previous
import functools

import jax
import jax.numpy as jnp
from jax import lax
from jax.experimental import pallas as pl
from jax.experimental.pallas import tpu as pltpu
from jax.experimental.pallas import tpu_sc as plsc

DIM = 32
IDX_PER_DMA = 128
CHUNK = 512
K = CHUNK // IDX_PER_DMA
LANES = 16


def _gather_kernel(seq, batch):
    n_chunks = seq
    assert n_chunks % 2 == 0
    n_pairs = n_chunks // 2
    mesh = plsc.VectorSubcoreMesh(core_axis_name="c", subcore_axis_name="s")
    nc = plsc.get_sparse_core_info().num_cores

    @functools.partial(
        pl.kernel,
        mesh=mesh,
        out_type=jax.ShapeDtypeStruct((seq, DIM, batch), jnp.float32),
        compiler_params=pltpu.CompilerParams(
            use_tc_tiling_on_sc=False, needs_layout_passes=False
        ),
        scratch_types=[
            pltpu.VMEM((seq, CHUNK), jnp.int32),
            pltpu.VMEM((CHUNK, DIM), jnp.float32),
            pltpu.VMEM((CHUNK, DIM), jnp.float32),
            pltpu.VMEM((DIM, CHUNK), jnp.float32),
            pltpu.VMEM((DIM, CHUNK), jnp.float32),
            pltpu.SemaphoreType.DMA,
            pltpu.SemaphoreType.DMA,
            pltpu.SemaphoreType.DMA,
            pltpu.SemaphoreType.DMA,
            pltpu.SemaphoreType.DMA,
        ],
    )
    def k(table_hbm, xt_hbm, out_hbm, xbuf, rows0, rows1, t0, t1,
          xsem, gsem0, gsem1, osem0, osem1):
        wid = lax.axis_index("s") * nc + lax.axis_index("c")
        b0 = pl.multiple_of(wid * CHUNK, CHUNK)
        lane = lax.iota(jnp.int32, LANES)
        diag_cols = [(lane + d0) & (DIM - 1) for d0 in range(DIM)]

        pltpu.async_copy(xt_hbm.at[:, pl.ds(b0, CHUNK)], xbuf, xsem).wait()

        def fire_gathers(s, rowsbuf, sem):
            for j in range(K):
                pltpu.async_copy(
                    table_hbm.at[xbuf.at[s, pl.ds(j * IDX_PER_DMA, IDX_PER_DMA)]],
                    rowsbuf.at[pl.ds(j * IDX_PER_DMA, IDX_PER_DMA)],
                    sem,
                )

        def wait_gathers(rowsbuf, sem):
            pltpu.make_async_copy(
                out_hbm.at[0, :, pl.ds(0, CHUNK)], rowsbuf, sem
            ).wait()

        def transpose(rowsbuf, tbuf):
            def g_body(g, _):
                rows16 = lane + g * LANES
                for d0 in range(DIM):
                    cols = diag_cols[d0]
                    vals = plsc.load_gather(rowsbuf, [rows16, cols])
                    plsc.store_scatter(tbuf, [cols, rows16], vals)
                return 0

            lax.fori_loop(0, CHUNK // LANES, g_body, 0, unroll=2)

        def fire_out(s, tbuf, sem):
            pltpu.async_copy(tbuf, out_hbm.at[s, :, pl.ds(b0, CHUNK)], sem)

        def wait_out(tbuf, sem):
            pltpu.make_async_copy(
                tbuf, out_hbm.at[0, :, pl.ds(0, CHUNK)], sem
            ).wait()

        fire_gathers(0, rows0, gsem0)
        fire_gathers(1, rows1, gsem1)

        wait_gathers(rows0, gsem0)
        transpose(rows0, t0)
        fire_out(0, t0, osem0)
        fire_gathers(2, rows0, gsem0)
        wait_gathers(rows1, gsem1)
        transpose(rows1, t1)
        fire_out(1, t1, osem1)
        fire_gathers(3, rows1, gsem1)

        def body(t, _):
            a = 2 * t
            wait_gathers(rows0, gsem0)
            wait_out(t0, osem0)
            transpose(rows0, t0)
            fire_out(a, t0, osem0)
            fire_gathers(a + 2, rows0, gsem0)
            wait_gathers(rows1, gsem1)
            wait_out(t1, osem1)
            transpose(rows1, t1)
            fire_out(a + 1, t1, osem1)
            fire_gathers(a + 3, rows1, gsem1)
            return 0

        lax.fori_loop(1, n_pairs - 1, body, 0, unroll=False)

        last = n_chunks - 2
        wait_gathers(rows0, gsem0)
        wait_out(t0, osem0)
        transpose(rows0, t0)
        fire_out(last, t0, osem0)
        wait_gathers(rows1, gsem1)
        wait_out(t1, osem1)
        transpose(rows1, t1)
        fire_out(last + 1, t1, osem1)
        wait_out(t0, osem0)
        wait_out(t1, osem1)

    return k


def kernel(x, y, embedding):
    b, s = x.shape
    out_p = _gather_kernel(s, b)(embedding, x.T.astype(jnp.int32))
    return jnp.transpose(out_p, (2, 0, 1))
